# Initial kernel scaffold; baseline (speedup 1.0000x reference)
import functools

import jax
import jax.numpy as jnp
from jax import lax
from jax.experimental import pallas as pl
from jax.experimental.pallas import tpu as pltpu
from jax.experimental.pallas import tpu_sc as plsc

NN = 10000
NE = 320000
NG = 64
NH = 64
NV = 16
XD = 4

NW = 32
PW = NE // NW
CH = 80
NCH = PW // CH
NPS = NN // 16

BE = 2560
GE = NE // BE
BN = 2000
GN = NN // BN

F32 = jnp.float32


def _sig(x):
    return 1.0 / (1.0 + jnp.exp(-x))


def _silu(x):
    return x * _sig(x)


def _ln(x, g, b):
    m = jnp.mean(x, axis=-1, keepdims=True)
    v = jnp.mean((x - m) ** 2, axis=-1, keepdims=True)
    return (x - m) * lax.rsqrt(v + 1e-5) * g + b


def _dot(a, b):
    return jnp.dot(a, b, preferred_element_type=F32)


def _rbf_parts(xs, xd):
    diff = xd - xs
    d2 = jnp.sum(diff * diff, axis=-1, keepdims=True)
    dist = jnp.sqrt(d2 + 1e-8)
    inv = 1.0 / dist
    mu = lax.broadcasted_iota(F32, (1, 16), 1) * (20.0 / 15.0)
    d = jnp.exp(-((dist - mu) ** 2) * (1.0 / (2.0 * 1.25 * 1.25)))
    xdp = [diff[:, k:k + 1] * inv for k in range(3)]
    return xdp, d


def _gvp_tail(z, Vh, wu, gate):
    s = _silu(z)
    mus = [_dot(h, wu) for h in Vh]
    if gate:
        gn = jnp.sqrt(mus[0] ** 2 + mus[1] ** 2 + mus[2] ** 2 + 1e-8)
        gt = _sig(gn)
        mus = [m * gt for m in mus]
    return s, mus


def _vn(Vh):
    return jnp.sqrt(Vh[0] ** 2 + Vh[1] ** 2 + Vh[2] ** 2 + 1e-8)


def _full_spec(shape):
    nd = len(shape)
    return pl.BlockSpec(shape, lambda i, _n=nd: (0,) * _n)


def _row_spec(block, width):
    return pl.BlockSpec((block, width), lambda i: (i, 0))


def _make_node_emb():
    def body(a, c, nbi, t, w1a, w1t, w1c, b1, w2, b2, g, bl, out):
        oh = (nbi[...] == lax.broadcasted_iota(jnp.int32, (BN, NG), 1))
        tn = _dot(oh.astype(F32), t[...])
        h = _silu(_dot(a[...], w1a[...]) + tn * w1t[...] +
                  _dot(c[...], w1c[...]) + b1[...])
        h = _silu(_dot(h, w2[...]) + b2[...])
        out[...] = _ln(h, g[...], bl[...])

    return pl.pallas_call(
        body,
        grid=(GN,),
        in_specs=[
            _row_spec(BN, 16), _row_spec(BN, 6), _row_spec(BN, 1),
            _full_spec((NG, 1)),
            _full_spec((16, NH)), _full_spec((1, NH)), _full_spec((6, NH)),
            _full_spec((1, NH)), _full_spec((NH, NH)), _full_spec((1, NH)),
            _full_spec((1, NH)), _full_spec((1, NH)),
        ],
        out_specs=_row_spec(BN, NH),
        out_shape=jax.ShapeDtypeStruct((NN, NH), F32),
    )


def _make_edge_emb():
    def body(et, w1, b1, w2, b2, g, bl, out):
        h = _silu(_dot(et[...], w1[...]) + b1[...])
        h = _silu(_dot(h, w2[...]) + b2[...])
        out[...] = _ln(h, g[...], bl[...])

    return pl.pallas_call(
        body,
        grid=(GE,),
        in_specs=[
            _row_spec(BE, 5),
            _full_spec((5, NH)), _full_spec((1, NH)),
            _full_spec((NH, NH)), _full_spec((1, NH)),
            _full_spec((1, NH)), _full_spec((1, NH)),
        ],
        out_specs=_row_spec(BE, NH),
        out_shape=jax.ShapeDtypeStruct((NE, NH), F32),
    )


def _make_edge_msg(has_v):
    def body(*refs):
        i = iter(refs)
        ss, sd, e, xs, xd = next(i), next(i), next(i), next(i), next(i)
        vs = next(i) if has_v else None
        wh1, wu1 = next(i), next(i)
        w1a, w1b, w1e, w1d, w1v, b1 = (next(i) for _ in range(6))
        wh2, wu2, w2s, w2v, b2 = (next(i) for _ in range(5))
        wh3, wu3, w3s, w3v, b3 = (next(i) for _ in range(5))
        oms, omv = next(i), next(i)

        xdp, d = _rbf_parts(xs[...], xd[...])
        wh1m = wh1[...]
        if has_v:
            vsr = vs[...]
            Vh = [_dot(vsr[:, 16 * p:16 * p + 16], wh1m[0:16, :]) +
                  xdp[p] * wh1m[16:17, :] for p in range(3)]
        else:
            Vh = [xdp[p] * wh1m[16:17, :] for p in range(3)]
        z = (_dot(ss[...], w1a[...]) + _dot(sd[...], w1b[...]) +
             _dot(e[...], w1e[...]) + _dot(d, w1d[...]) +
             _dot(_vn(Vh), w1v[...]) + b1[...])
        s, V = _gvp_tail(z, Vh, wu1[...], True)

        Vh = [_dot(v, wh2[...]) for v in V]
        z = _dot(s, w2s[...]) + _dot(_vn(Vh), w2v[...]) + b2[...]
        s, V = _gvp_tail(z, Vh, wu2[...], True)

        Vh = [_dot(v, wh3[...]) for v in V]
        z = _dot(s, w3s[...]) + _dot(_vn(Vh), w3v[...]) + b3[...]
        s, V = _gvp_tail(z, Vh, wu3[...], True)

        oms[...] = s
        omv[...] = jnp.concatenate(V, axis=1)

    data = [_row_spec(BE, NH), _row_spec(BE, NH), _row_spec(BE, NH),
            _row_spec(BE, XD), _row_spec(BE, XD)]
    if has_v:
        data.append(_row_spec(BE, 3 * NV))
    wspecs = [
        _full_spec((17, 17)), _full_spec((17, NV)),
        _full_spec((NH, NH)), _full_spec((NH, NH)), _full_spec((NH, NH)),
        _full_spec((16, NH)), _full_spec((17, NH)), _full_spec((1, NH)),
        _full_spec((NV, NV)), _full_spec((NV, NV)),
        _full_spec((NH, NH)), _full_spec((NV, NH)), _full_spec((1, NH)),
        _full_spec((NV, NV)), _full_spec((NV, NV)),
        _full_spec((NH, NH)), _full_spec((NV, NH)), _full_spec((1, NH)),
    ]
    return pl.pallas_call(
        body,
        grid=(GE,),
        in_specs=data + wspecs,
        out_specs=[_row_spec(BE, NH), _row_spec(BE, 3 * NV)],
        out_shape=[jax.ShapeDtypeStruct((NE, NH), F32),
                   jax.ShapeDtypeStruct((NE, 3 * NV), F32)],
    )


def _upd_gvp(s, V, wh, wu, ws, wv, b, gate=True):
    Vh = [_dot(v, wh) for v in V]
    z = _dot(s, ws) + _dot(_vn(Vh), wv) + b
    return _gvp_tail(z, Vh, wu, gate)


def _make_node_upd(pos, head):
    def body(*refs):
        i = iter(refs)
        s0, v0, ps, pv = next(i), next(i), next(i), next(i)
        g1, bl1, g2, bl2 = (next(i) for _ in range(4))
        uw = [[next(i) for _ in range(5)] for _ in range(3)]
        if pos:
            x = next(i)
            pw = [[next(i) for _ in range(5)] for _ in range(2)]
            wh3, wu3 = next(i), next(i)
        if head:
            hw1, hb1, hw2, hb2 = (next(i) for _ in range(4))
        os_, ov = next(i), next(i)
        if pos:
            ox = next(i)
        if head:
            ol = next(i)

        psr, pvr = ps[...], pv[...]
        aggs = (psr[0] + psr[1]) * (1.0 / 100.0)
        aggv = (pvr[0] + pvr[1]) * (1.0 / 100.0)
        s = _ln(s0[...] + aggs, g1[...], bl1[...])
        v0r = v0[...]
        V = [v0r[:, 16 * p:16 * p + 16] + aggv[:, 16 * p:16 * p + 16]
             for p in range(3)]
        us, uv = s, V
        for w in uw:
            us, uv = _upd_gvp(us, uv, w[0][...], w[1][...], w[2][...],
                              w[3][...], w[4][...])
        s2 = _ln(s + us, g2[...], bl2[...])
        V2 = [V[p] + uv[p] for p in range(3)]
        os_[...] = s2
        ov[...] = jnp.concatenate(V2, axis=1)

        if pos:
            qs, qv = s2, V2
            for w in pw:
                qs, qv = _upd_gvp(qs, qv, w[0][...], w[1][...], w[2][...],
                                  w[3][...], w[4][...])
            Vh = [_dot(v, wh3[...]) for v in qv]
            mus = [_dot(h, wu3[...]) for h in Vh]
            xr = x[...]
            ox[...] = jnp.concatenate(
                [xr[:, p:p + 1] + mus[p] for p in range(3)] + [xr[:, 3:4]],
                axis=1)
        if head:
            hh = _silu(_dot(s2, hw1[...]) + hb1[...])
            ol[...] = _dot(hh, hw2[...]) + hb2[...]

    specs = [_row_spec(BN, NH), _row_spec(BN, 3 * NV),
             pl.BlockSpec((2, BN, NH), lambda i: (0, i, 0)),
             pl.BlockSpec((2, BN, 3 * NV), lambda i: (0, i, 0)),
             _full_spec((1, NH)), _full_spec((1, NH)),
             _full_spec((1, NH)), _full_spec((1, NH))]
    gvp_w = [_full_spec((NV, NV)), _full_spec((NV, NV)),
             _full_spec((NH, NH)), _full_spec((NV, NH)), _full_spec((1, NH))]
    specs += gvp_w * 3
    if pos:
        specs += [_row_spec(BN, XD)] + gvp_w * 2
        specs += [_full_spec((NV, NV)), _full_spec((NV, 1))]
    if head:
        specs += [_full_spec((NH, NH)), _full_spec((1, NH)),
                  _full_spec((NH, 22)), _full_spec((1, 22))]
    outs = [_row_spec(BN, NH), _row_spec(BN, 3 * NV)]
    oshapes = [jax.ShapeDtypeStruct((NN, NH), F32),
               jax.ShapeDtypeStruct((NN, 3 * NV), F32)]
    if pos:
        outs.append(_row_spec(BN, XD))
        oshapes.append(jax.ShapeDtypeStruct((NN, XD), F32))
    if head:
        outs.append(_row_spec(BN, 22))
        oshapes.append(jax.ShapeDtypeStruct((NN, 22), F32))
    return pl.pallas_call(body, grid=(GN,), in_specs=specs,
                          out_specs=outs, out_shape=oshapes)


def _make_edge_upd(head):
    def body(*refs):
        i = iter(refs)
        ss, sd, e, xs, xd = (next(i) for _ in range(5))
        wa, wb, we, wd, b1, w2, b2, g, bl = (next(i) for _ in range(9))
        if head:
            hw1, hb1, hw2, hb2 = (next(i) for _ in range(4))
        oe = next(i)
        if head:
            ol = next(i)
        _, d = _rbf_parts(xs[...], xd[...])
        er = e[...]
        h = _silu(_dot(ss[...], wa[...]) + _dot(sd[...], wb[...]) +
                  _dot(er, we[...]) + _dot(d, wd[...]) + b1[...])
        h = _silu(_dot(h, w2[...]) + b2[...])
        en = _ln(er + h, g[...], bl[...])
        oe[...] = en
        if head:
            hh = _silu(_dot(en, hw1[...]) + hb1[...])
            ol[...] = _dot(hh, hw2[...]) + hb2[...]

    specs = [_row_spec(BE, NH), _row_spec(BE, NH), _row_spec(BE, NH),
             _row_spec(BE, XD), _row_spec(BE, XD),
             _full_spec((NH, NH)), _full_spec((NH, NH)), _full_spec((NH, NH)),
             _full_spec((16, NH)), _full_spec((1, NH)),
             _full_spec((NH, NH)), _full_spec((1, NH)),
             _full_spec((1, NH)), _full_spec((1, NH))]
    outs = [_row_spec(BE, NH)]
    oshapes = [jax.ShapeDtypeStruct((NE, NH), F32)]
    if head:
        specs += [_full_spec((NH, NH)), _full_spec((1, NH)),
                  _full_spec((NH, 5)), _full_spec((1, 5))]
        outs.append(_row_spec(BE, 5))
        oshapes.append(jax.ShapeDtypeStruct((NE, 5), F32))
    return pl.pallas_call(body, grid=(GE,), in_specs=specs,
                          out_specs=outs, out_shape=oshapes)


_MESH = plsc.VectorSubcoreMesh(core_axis_name="c", subcore_axis_name="s")


def _make_gather(with_v, with_x):
    out_type = [jax.ShapeDtypeStruct((NE, NH), F32),
                jax.ShapeDtypeStruct((NE, NH), F32)]
    if with_v:
        out_type.append(jax.ShapeDtypeStruct((NE, 3 * NV), F32))
    if with_x:
        out_type.append(jax.ShapeDtypeStruct((NE, XD), F32))
        out_type.append(jax.ShapeDtypeStruct((NE, XD), F32))
    scratch = [pltpu.VMEM((PW,), jnp.int32), pltpu.VMEM((PW,), jnp.int32),
               pltpu.VMEM((CH, NH), F32), pltpu.VMEM((CH, NH), F32)]
    if with_v:
        scratch.append(pltpu.VMEM((CH, 3 * NV), F32))
    if with_x:
        scratch.append(pltpu.VMEM((CH, XD), F32))
        scratch.append(pltpu.VMEM((CH, XD), F32))
    scratch.append(pltpu.SemaphoreType.DMA)

    @functools.partial(pl.kernel, mesh=_MESH, out_type=out_type,
                       scratch_types=scratch)
    def k(*refs):
        i = iter(refs)
        src_h, dst_h, s_h = next(i), next(i), next(i)
        v_h = next(i) if with_v else None
        x_h = next(i) if with_x else None
        o_ss, o_sd = next(i), next(i)
        o_v = next(i) if with_v else None
        if with_x:
            o_xs, o_xd = next(i), next(i)
        isrc, idst, bs1, bs2 = next(i), next(i), next(i), next(i)
        bv = next(i) if with_v else None
        if with_x:
            bx1, bx2 = next(i), next(i)
        sem = next(i)

        wid = lax.axis_index("s") * 2 + lax.axis_index("c")
        pltpu.sync_copy(src_h.at[wid], isrc)
        pltpu.sync_copy(dst_h.at[wid], idst)

        def it(j, carry):
            off = wid * PW + j * CH
            ia = isrc.at[pl.ds(j * CH, CH)]
            ib = idst.at[pl.ds(j * CH, CH)]
            cps = [pltpu.async_copy(s_h.at[ia], bs1, sem),
                   pltpu.async_copy(s_h.at[ib], bs2, sem)]
            if with_v:
                cps.append(pltpu.async_copy(v_h.at[ia], bv, sem))
            if with_x:
                cps.append(pltpu.async_copy(x_h.at[ia], bx1, sem))
                cps.append(pltpu.async_copy(x_h.at[ib], bx2, sem))
            for cp in cps:
                cp.wait()
            pltpu.sync_copy(bs1, o_ss.at[pl.ds(off, CH)])
            pltpu.sync_copy(bs2, o_sd.at[pl.ds(off, CH)])
            if with_v:
                pltpu.sync_copy(bv, o_v.at[pl.ds(off, CH)])
            if with_x:
                pltpu.sync_copy(bx1, o_xs.at[pl.ds(off, CH)])
                pltpu.sync_copy(bx2, o_xd.at[pl.ds(off, CH)])
            return carry

        lax.fori_loop(0, NCH, it, 0)

    return k


def _make_scatter():
    out_type = [jax.ShapeDtypeStruct((2, NN, NH), F32),
                jax.ShapeDtypeStruct((2, NN, 3 * NV), F32)]
    scratch = [pltpu.VMEM((NCH, CH), jnp.int32),
               pltpu.VMEM((CH, NH), F32), pltpu.VMEM((CH, 3 * NV), F32),
               pltpu.VMEM_SHARED((NN, NH), F32),
               pltpu.VMEM_SHARED((NN, 3 * NV), F32)]

    @functools.partial(pl.kernel, mesh=_MESH, out_type=out_type,
                       scratch_types=scratch)
    def k(dst_h, ms_h, mv_h, zs_h, zv_h, o_s, o_v, idxb, bms, bmv,
          accs, accv):
        cid = lax.axis_index("c")
        sid = lax.axis_index("s")
        wid = sid * 2 + cid
        pltpu.sync_copy(zs_h.at[pl.ds(sid * NPS, NPS)],
                        accs.at[pl.ds(sid * NPS, NPS)])
        pltpu.sync_copy(zv_h.at[pl.ds(sid * NPS, NPS)],
                        accv.at[pl.ds(sid * NPS, NPS)])
        plsc.subcore_barrier()
        pltpu.sync_copy(dst_h.at[wid], idxb)

        def it(j, carry):
            off = wid * PW + j * CH
            pltpu.sync_copy(ms_h.at[pl.ds(off, CH)], bms)
            pltpu.sync_copy(mv_h.at[pl.ds(off, CH)], bmv)
            pltpu.sync_copy(bms, accs.at[idxb.at[j]], add=True)
            pltpu.sync_copy(bmv, accv.at[idxb.at[j]], add=True)
            return carry

        lax.fori_loop(0, NCH, it, 0)
        plsc.subcore_barrier()
        pltpu.sync_copy(accs.at[pl.ds(sid * NPS, NPS)],
                        o_s.at[cid, pl.ds(sid * NPS, NPS)])
        pltpu.sync_copy(accv.at[pl.ds(sid * NPS, NPS)],
                        o_v.at[cid, pl.ds(sid * NPS, NPS)])

    return k


_node_emb = _make_node_emb()
_edge_emb = _make_edge_emb()
_edge_msg0 = _make_edge_msg(False)
_edge_msg = _make_edge_msg(True)
_node_upd = _make_node_upd(False, False)
_node_upd_pos = _make_node_upd(True, False)
_node_upd_pos_head = _make_node_upd(True, True)
_edge_upd = _make_edge_upd(False)
_edge_upd_head = _make_edge_upd(True)
_gather_sx = _make_gather(False, True)
_gather_sv = _make_gather(True, False)
_gather_svx = _make_gather(True, True)
_scatter = _make_scatter()


def _b2(b):
    return b.reshape(1, -1)


def _msg_weights(conv):
    g1, g2, g3 = conv["msg"]
    w1 = g1["Ws"]["W"]
    out = [g1["Wh"], g1["Wu"],
           w1[0:64], w1[64:128], w1[128:192], w1[192:208], w1[208:225],
           _b2(g1["Ws"]["b"])]
    for g in (g2, g3):
        w = g["Ws"]["W"]
        out += [g["Wh"], g["Wu"], w[0:64], w[64:80], _b2(g["Ws"]["b"])]
    return out


def _gvp5(g):
    w = g["Ws"]["W"]
    return [g["Wh"], g["Wu"], w[0:64], w[64:80], _b2(g["Ws"]["b"])]


def _upd_weights(conv):
    out = [_b2(conv["ln1"]["g"]), _b2(conv["ln1"]["b"]),
           _b2(conv["ln2"]["g"]), _b2(conv["ln2"]["b"])]
    for g in conv["upd"]:
        out += _gvp5(g)
    return out


def _eupd_weights(p):
    w = p["l1"]["W"]
    return [w[0:64], w[64:128], w[128:192], w[192:208], _b2(p["l1"]["b"]),
            p["l2"]["W"], _b2(p["l2"]["b"]),
            _b2(p["ln"]["g"]), _b2(p["ln"]["b"])]


def kernel(a_t, c_t, x_t, e_t, t, edge_index, node_batch_idx,
           upper_edge_mask, params):
    src2 = edge_index[0].reshape(NW, PW)
    dst2 = edge_index[1].reshape(NW, PW)
    dst3 = edge_index[1].reshape(NW, NCH, CH)
    x0 = jnp.pad(x_t, ((0, 0), (0, 1)))
    nbi = node_batch_idx.reshape(NN, 1)
    t2 = t.reshape(NG, 1)
    zs = jnp.zeros((NN, NH), F32)
    zv = jnp.zeros((NN, 3 * NV), F32)

    pe = params["scalar_emb"]
    w1 = pe["l1"]["W"]
    s = _node_emb(a_t, c_t, nbi, t2, w1[0:16], w1[16:17], w1[17:23],
                  _b2(pe["l1"]["b"]), pe["l2"]["W"], _b2(pe["l2"]["b"]),
                  _b2(pe["ln"]["g"]), _b2(pe["ln"]["b"]))
    pee = params["edge_emb"]
    e = _edge_emb(e_t, pee["l1"]["W"], _b2(pee["l1"]["b"]), pee["l2"]["W"],
                  _b2(pee["l2"]["b"]), _b2(pee["ln"]["g"]), _b2(pee["ln"]["b"]))

    convs = params["convs"]
    pos_w = []
    for g in params["pos_upd"][:2]:
        pos_w += _gvp5(g)
    pos_w += [params["pos_upd"][2]["Wh"], params["pos_upd"][2]["Wu"]]
    nh = params["node_head"]
    head_w = [nh["l1"]["W"], _b2(nh["l1"]["b"]), nh["l2"]["W"],
              _b2(nh["l2"]["b"])]
    eh = params["edge_head"]
    ehead_w = [eh["l1"]["W"], _b2(eh["l1"]["b"]), eh["l2"]["W"],
               _b2(eh["l2"]["b"])]

    ss, sd, xs, xd = _gather_sx(src2, dst2, s, x0)
    ms, mv = _edge_msg0(ss, sd, e, xs, xd, *_msg_weights(convs[0]))
    ps, pv = _scatter(dst3, ms, mv, zs, zv)
    s, V = _node_upd(s, zv, ps, pv, *_upd_weights(convs[0]))

    ss, sd, vs = _gather_sv(src2, dst2, s, V)
    ms, mv = _edge_msg(ss, sd, e, xs, xd, vs, *_msg_weights(convs[1]))
    ps, pv = _scatter(dst3, ms, mv, zs, zv)
    s, V, x1 = _node_upd_pos(s, V, ps, pv, *_upd_weights(convs[1]), x0,
                             *pos_w)

    ss, sd, vs, xs, xd = _gather_svx(src2, dst2, s, V, x1)
    e = _edge_upd(ss, sd, e, xs, xd, *_eupd_weights(params["edge_upd"]))[0]

    ms, mv = _edge_msg(ss, sd, e, xs, xd, vs, *_msg_weights(convs[2]))
    ps, pv = _scatter(dst3, ms, mv, zs, zv)
    s, V = _node_upd(s, V, ps, pv, *_upd_weights(convs[2]))

    ss, sd, vs = _gather_sv(src2, dst2, s, V)
    ms, mv = _edge_msg(ss, sd, e, xs, xd, vs, *_msg_weights(convs[3]))
    ps, pv = _scatter(dst3, ms, mv, zs, zv)
    s, V, x2, node_logits = _node_upd_pos_head(
        s, V, ps, pv, *_upd_weights(convs[3]), x1, *pos_w, *head_w)

    ss, sd, xs, xd = _gather_sx(src2, dst2, s, x2)
    e, edge_logits = _edge_upd_head(ss, sd, e, xs, xd,
                                    *_eupd_weights(params["edge_upd"]),
                                    *ehead_w)

    return node_logits, edge_logits, x2[:, :3]


# speedup vs baseline: 10.5708x; 10.5708x over previous
import functools

import jax
import jax.numpy as jnp
from jax import lax
from jax.experimental import pallas as pl
from jax.experimental.pallas import tpu as pltpu
from jax.experimental.pallas import tpu_sc as plsc

NN = 10000
NNP = 10240
NE = 320000
NG = 64
NH = 64
NV = 16
NU = 128

NW = 32
PW = NE // NW
CH = 80
NCH = PW // CH
NPS = NNP // 16

BE = 2560
GE = NE // BE
BN = 2000
GN = NN // BN

F32 = jnp.float32


def _sig(x):
    return 1.0 / (1.0 + jnp.exp(-x))


def _silu(x):
    return x * _sig(x)


def _ln(x, g, b):
    m = jnp.mean(x, axis=-1, keepdims=True)
    v = jnp.mean((x - m) ** 2, axis=-1, keepdims=True)
    return (x - m) * lax.rsqrt(v + 1e-5) * g + b


def _dot(a, b):
    return jnp.dot(a, b, preferred_element_type=F32)


def _rbf_parts(xs, xd):
    diff = xd - xs
    d2 = jnp.sum(diff * diff, axis=-1, keepdims=True)
    dist = jnp.sqrt(d2 + 1e-8)
    inv = 1.0 / dist
    mu = (lax.broadcasted_iota(jnp.int32, (1, 16), 1).astype(F32)
          * (20.0 / 15.0))
    d = jnp.exp(-((dist - mu) ** 2) * (1.0 / (2.0 * 1.25 * 1.25)))
    xdp = [diff[:, k:k + 1] * inv for k in range(3)]
    return xdp, d


def _gvp_tail(z, Vh, wu, gate):
    s = _silu(z)
    mus = [_dot(h, wu) for h in Vh]
    if gate:
        gn = jnp.sqrt(mus[0] ** 2 + mus[1] ** 2 + mus[2] ** 2 + 1e-8)
        gt = _sig(gn)
        mus = [m * gt for m in mus]
    return s, mus


def _vn(Vh):
    return jnp.sqrt(Vh[0] ** 2 + Vh[1] ** 2 + Vh[2] ** 2 + 1e-8)


def _upd_gvp(s, V, wh, wu, ws, wv, b, gate=True):
    Vh = [_dot(v, wh) for v in V]
    z = _dot(s, ws) + _dot(_vn(Vh), wv) + b
    return _gvp_tail(z, Vh, wu, gate)


def _full_spec(shape):
    nd = len(shape)
    return pl.BlockSpec(shape, lambda i, _n=nd: (0,) * _n)


def _row_spec(block, width):
    return pl.BlockSpec((block, width), lambda i: (i, 0))


def _u_parts(u):
    s = u[:, 0:NH]
    V = [u[:, NH + 16 * p:NH + 16 * p + 16] for p in range(3)]
    x = u[:, 112:115]
    return s, V, x


def _pack_u(s, V, x):
    z = jnp.zeros((s.shape[0], 13), F32)
    return jnp.concatenate([s] + V + [x, z], axis=1)


def _make_node_emb():
    def body(a, c, nbi, t, x, w1a, w1t, w1c, b1, w2, b2, g, bl, out):
        oh = (nbi[...] == lax.broadcasted_iota(jnp.int32, (BN, NG), 1))
        tn = _dot(oh.astype(F32), t[...])
        h = _silu(_dot(a[...], w1a[...]) + tn * w1t[...] +
                  _dot(c[...], w1c[...]) + b1[...])
        h = _silu(_dot(h, w2[...]) + b2[...])
        s = _ln(h, g[...], bl[...])
        zv = jnp.zeros((BN, 16), F32)
        out[...] = _pack_u(s, [zv, zv, zv], x[...])

    return pl.pallas_call(
        body,
        grid=(GN,),
        in_specs=[
            _row_spec(BN, 16), _row_spec(BN, 6), _row_spec(BN, 1),
            _full_spec((NG, 1)), _row_spec(BN, 3),
            _full_spec((16, NH)), _full_spec((1, NH)), _full_spec((6, NH)),
            _full_spec((1, NH)), _full_spec((NH, NH)), _full_spec((1, NH)),
            _full_spec((1, NH)), _full_spec((1, NH)),
        ],
        out_specs=_row_spec(BN, NU),
        out_shape=jax.ShapeDtypeStruct((NN, NU), F32),
    )


def _make_edge_emb():
    def body(et, w1, b1, w2, b2, g, bl, out):
        h = _silu(_dot(et[...], w1[...]) + b1[...])
        h = _silu(_dot(h, w2[...]) + b2[...])
        out[...] = _ln(h, g[...], bl[...])

    return pl.pallas_call(
        body,
        grid=(GE,),
        in_specs=[
            _row_spec(BE, 5),
            _full_spec((5, NH)), _full_spec((1, NH)),
            _full_spec((NH, NH)), _full_spec((1, NH)),
            _full_spec((1, NH)), _full_spec((1, NH)),
        ],
        out_specs=_row_spec(BE, NH),
        out_shape=jax.ShapeDtypeStruct((NE, NH), F32),
    )


def _make_edge_msg():
    def body(*refs):
        i = iter(refs)
        us, ud, e = next(i), next(i), next(i)
        wh1, wu1 = next(i), next(i)
        w1a, w1b, w1e, w1d, w1v, b1 = (next(i) for _ in range(6))
        wh2, wu2, w2s, w2v, b2 = (next(i) for _ in range(5))
        wh3, wu3, w3s, w3v, b3 = (next(i) for _ in range(5))
        om = next(i)

        usr, udr = us[...], ud[...]
        ss, vsr, xs = _u_parts(usr)
        sd, _, xd = _u_parts(udr)
        xdp, d = _rbf_parts(xs, xd)
        wh1m = wh1[...]
        Vh = [_dot(vsr[p], wh1m[0:16, :]) + xdp[p] * wh1m[16:17, :]
              for p in range(3)]
        z = (_dot(ss, w1a[...]) + _dot(sd, w1b[...]) +
             _dot(e[...], w1e[...]) + _dot(d, w1d[...]) +
             _dot(_vn(Vh), w1v[...]) + b1[...])
        s, V = _gvp_tail(z, Vh, wu1[...], True)

        Vh = [_dot(v, wh2[...]) for v in V]
        z = _dot(s, w2s[...]) + _dot(_vn(Vh), w2v[...]) + b2[...]
        s, V = _gvp_tail(z, Vh, wu2[...], True)

        Vh = [_dot(v, wh3[...]) for v in V]
        z = _dot(s, w3s[...]) + _dot(_vn(Vh), w3v[...]) + b3[...]
        s, V = _gvp_tail(z, Vh, wu3[...], True)

        om[...] = jnp.concatenate([s] + V + [jnp.zeros((BE, 16), F32)],
                                  axis=1)

    wspecs = [
        _full_spec((17, 17)), _full_spec((17, NV)),
        _full_spec((NH, NH)), _full_spec((NH, NH)), _full_spec((NH, NH)),
        _full_spec((16, NH)), _full_spec((17, NH)), _full_spec((1, NH)),
        _full_spec((NV, NV)), _full_spec((NV, NV)),
        _full_spec((NH, NH)), _full_spec((NV, NH)), _full_spec((1, NH)),
        _full_spec((NV, NV)), _full_spec((NV, NV)),
        _full_spec((NH, NH)), _full_spec((NV, NH)), _full_spec((1, NH)),
    ]
    return pl.pallas_call(
        body,
        grid=(GE,),
        in_specs=[_row_spec(BE, NU), _row_spec(BE, NU),
                  _row_spec(BE, NH)] + wspecs,
        out_specs=_row_spec(BE, NU),
        out_shape=jax.ShapeDtypeStruct((NE, NU), F32),
    )


def _make_node_upd(pos, head):
    def body(*refs):
        i = iter(refs)
        u0, pp = next(i), next(i)
        g1, bl1, g2, bl2 = (next(i) for _ in range(4))
        uw = [[next(i) for _ in range(5)] for _ in range(3)]
        if pos:
            pw = [[next(i) for _ in range(5)] for _ in range(2)]
            wh3, wu3 = next(i), next(i)
        if head:
            hw1, hb1, hw2, hb2 = (next(i) for _ in range(4))
        ou = next(i)
        ol = next(i) if head else None

        u0r = u0[...]
        s0, V0, x = _u_parts(u0r)
        ppr = pp[...]
        agg = (ppr[0] + ppr[1]) * (1.0 / 100.0)
        s = _ln(s0 + agg[:, 0:NH], g1[...], bl1[...])
        V = [V0[p] + agg[:, NH + 16 * p:NH + 16 * p + 16] for p in range(3)]
        us, uv = s, V
        for w in uw:
            us, uv = _upd_gvp(us, uv, w[0][...], w[1][...], w[2][...],
                              w[3][...], w[4][...])
        s2 = _ln(s + us, g2[...], bl2[...])
        V2 = [V[p] + uv[p] for p in range(3)]

        if pos:
            qs, qv = s2, V2
            for w in pw:
                qs, qv = _upd_gvp(qs, qv, w[0][...], w[1][...], w[2][...],
                                  w[3][...], w[4][...])
            Vh = [_dot(v, wh3[...]) for v in qv]
            mus = [_dot(h, wu3[...]) for h in Vh]
            x = jnp.concatenate([x[:, p:p + 1] + mus[p] for p in range(3)],
                                axis=1)
        ou[...] = _pack_u(s2, V2, x)
        if head:
            hh = _silu(_dot(s2, hw1[...]) + hb1[...])
            ol[...] = _dot(hh, hw2[...]) + hb2[...]

    specs = [_row_spec(BN, NU),
             pl.BlockSpec((2, BN, NU), lambda i: (0, i, 0)),
             _full_spec((1, NH)), _full_spec((1, NH)),
             _full_spec((1, NH)), _full_spec((1, NH))]
    gvp_w = [_full_spec((NV, NV)), _full_spec((NV, NV)),
             _full_spec((NH, NH)), _full_spec((NV, NH)), _full_spec((1, NH))]
    specs += gvp_w * 3
    if pos:
        specs += gvp_w * 2
        specs += [_full_spec((NV, NV)), _full_spec((NV, 1))]
    if head:
        specs += [_full_spec((NH, NH)), _full_spec((1, NH)),
                  _full_spec((NH, 22)), _full_spec((1, 22))]
    outs = [_row_spec(BN, NU)]
    oshapes = [jax.ShapeDtypeStruct((NN, NU), F32)]
    if head:
        outs.append(_row_spec(BN, 22))
        oshapes.append(jax.ShapeDtypeStruct((NN, 22), F32))
    return pl.pallas_call(body, grid=(GN,), in_specs=specs,
                          out_specs=outs, out_shape=oshapes)


def _make_edge_upd(head):
    def body(*refs):
        i = iter(refs)
        us, ud, e = (next(i) for _ in range(3))
        wa, wb, we, wd, b1, w2, b2, g, bl = (next(i) for _ in range(9))
        if head:
            hw1, hb1, hw2, hb2 = (next(i) for _ in range(4))
        oe = next(i)
        ol = next(i) if head else None

        ss, _, xs = _u_parts(us[...])
        sd, _, xd = _u_parts(ud[...])
        _, d = _rbf_parts(xs, xd)
        er = e[...]
        h = _silu(_dot(ss, wa[...]) + _dot(sd, wb[...]) +
                  _dot(er, we[...]) + _dot(d, wd[...]) + b1[...])
        h = _silu(_dot(h, w2[...]) + b2[...])
        en = _ln(er + h, g[...], bl[...])
        oe[...] = en
        if head:
            hh = _silu(_dot(en, hw1[...]) + hb1[...])
            ol[...] = _dot(hh, hw2[...]) + hb2[...]

    specs = [_row_spec(BE, NU), _row_spec(BE, NU), _row_spec(BE, NH),
             _full_spec((NH, NH)), _full_spec((NH, NH)), _full_spec((NH, NH)),
             _full_spec((16, NH)), _full_spec((1, NH)),
             _full_spec((NH, NH)), _full_spec((1, NH)),
             _full_spec((1, NH)), _full_spec((1, NH))]
    outs = [_row_spec(BE, NH)]
    oshapes = [jax.ShapeDtypeStruct((NE, NH), F32)]
    if head:
        specs += [_full_spec((NH, NH)), _full_spec((1, NH)),
                  _full_spec((NH, 5)), _full_spec((1, 5))]
        outs.append(_row_spec(BE, 5))
        oshapes.append(jax.ShapeDtypeStruct((NE, 5), F32))
    return pl.pallas_call(body, grid=(GE,), in_specs=specs,
                          out_specs=outs, out_shape=oshapes)


def _sc_mesh():
    return plsc.VectorSubcoreMesh(core_axis_name="c", subcore_axis_name="s",
                                  num_cores=2)


def _make_gather():
    out_type = [jax.ShapeDtypeStruct((NE, NU), F32),
                jax.ShapeDtypeStruct((NE, NU), F32)]
    scratch = [pltpu.VMEM((PW,), jnp.int32), pltpu.VMEM((PW,), jnp.int32),
               pltpu.VMEM((CH, NU), F32), pltpu.VMEM((CH, NU), F32),
               pltpu.SemaphoreType.DMA]

    @functools.partial(pl.kernel, mesh=_sc_mesh(), out_type=out_type,
                       scratch_types=scratch)
    def k(src_h, dst_h, u_h, o_us, o_ud, isrc, idst, b1, b2, sem):
        wid = lax.axis_index("s") * 2 + lax.axis_index("c")
        pltpu.sync_copy(src_h.at[pl.ds(wid * PW, PW)], isrc)
        pltpu.sync_copy(dst_h.at[pl.ds(wid * PW, PW)], idst)

        def it(j, carry):
            off = wid * PW + j * CH
            ia = isrc.at[pl.ds(j * CH, CH)]
            ib = idst.at[pl.ds(j * CH, CH)]
            c1 = pltpu.async_copy(u_h.at[ia], b1, sem)
            c2 = pltpu.async_copy(u_h.at[ib], b2, sem)
            c1.wait()
            c2.wait()
            pltpu.sync_copy(b1, o_us.at[pl.ds(off, CH)])
            pltpu.sync_copy(b2, o_ud.at[pl.ds(off, CH)])
            return carry

        lax.fori_loop(0, NCH, it, 0)

    return k


def _make_scatter():
    out_type = [jax.ShapeDtypeStruct((2, NNP, NU), F32)]
    scratch = [pltpu.VMEM((NCH, CH), jnp.int32),
               pltpu.VMEM((CH, NU), F32),
               pltpu.VMEM_SHARED((NNP, NU), F32)]

    @functools.partial(pl.kernel, mesh=_sc_mesh(), out_type=out_type,
                       scratch_types=scratch)
    def k(dst_h, m_h, z_h, o_p, idxb, bm, acc):
        cid = lax.axis_index("c")
        sid = lax.axis_index("s")
        wid = sid * 2 + cid
        pltpu.sync_copy(z_h.at[pl.ds(sid * NPS, NPS)],
                        acc.at[pl.ds(sid * NPS, NPS)])
        pltpu.sync_copy(dst_h.at[wid], idxb)
        plsc.subcore_barrier()

        def it(j, carry):
            off = wid * PW + j * CH
            pltpu.sync_copy(m_h.at[pl.ds(off, CH)], bm)
            pltpu.sync_copy(bm, acc.at[idxb.at[j]], add=True)
            return carry

        lax.fori_loop(0, NCH, it, 0)
        plsc.subcore_barrier()
        pltpu.sync_copy(acc.at[pl.ds(sid * NPS, NPS)],
                        o_p.at[cid, pl.ds(sid * NPS, NPS)])

    return k


_node_emb = _make_node_emb()
_edge_emb = _make_edge_emb()
_edge_msg = _make_edge_msg()
_node_upd = _make_node_upd(False, False)
_node_upd_pos = _make_node_upd(True, False)
_node_upd_pos_head = _make_node_upd(True, True)
_edge_upd = _make_edge_upd(False)
_edge_upd_head = _make_edge_upd(True)

_sc_kernels = {}


def _sc_get(name, factory):
    if name not in _sc_kernels:
        _sc_kernels[name] = factory()
    return _sc_kernels[name]


def _gather(*args):
    return _sc_get("gather", _make_gather)(*args)


def _scatter(*args):
    return _sc_get("scatter", _make_scatter)(*args)


def _b2(b):
    return b.reshape(1, -1)


def _msg_weights(conv):
    g1, g2, g3 = conv["msg"]
    w1 = g1["Ws"]["W"]
    out = [g1["Wh"], g1["Wu"],
           w1[0:64], w1[64:128], w1[128:192], w1[192:208], w1[208:225],
           _b2(g1["Ws"]["b"])]
    for g in (g2, g3):
        w = g["Ws"]["W"]
        out += [g["Wh"], g["Wu"], w[0:64], w[64:80], _b2(g["Ws"]["b"])]
    return out


def _gvp5(g):
    w = g["Ws"]["W"]
    return [g["Wh"], g["Wu"], w[0:64], w[64:80], _b2(g["Ws"]["b"])]


def _upd_weights(conv):
    out = [_b2(conv["ln1"]["g"]), _b2(conv["ln1"]["b"]),
           _b2(conv["ln2"]["g"]), _b2(conv["ln2"]["b"])]
    for g in conv["upd"]:
        out += _gvp5(g)
    return out


def _eupd_weights(p):
    w = p["l1"]["W"]
    return [w[0:64], w[64:128], w[128:192], w[192:208], _b2(p["l1"]["b"]),
            p["l2"]["W"], _b2(p["l2"]["b"]),
            _b2(p["ln"]["g"]), _b2(p["ln"]["b"])]


def kernel(a_t, c_t, x_t, e_t, t, edge_index, node_batch_idx,
           upper_edge_mask, params):
    src1 = edge_index[0]
    dst1 = edge_index[1]
    dst3 = dst1.reshape(NW, NCH, CH)
    nbi = node_batch_idx.reshape(NN, 1)
    t2 = t.reshape(NG, 1)
    zp = jnp.zeros((NNP, NU), F32)

    pe = params["scalar_emb"]
    w1 = pe["l1"]["W"]
    u = _node_emb(a_t, c_t, nbi, t2, x_t, w1[0:16], w1[16:17], w1[17:23],
                  _b2(pe["l1"]["b"]), pe["l2"]["W"], _b2(pe["l2"]["b"]),
                  _b2(pe["ln"]["g"]), _b2(pe["ln"]["b"]))
    pee = params["edge_emb"]
    e = _edge_emb(e_t, pee["l1"]["W"], _b2(pee["l1"]["b"]), pee["l2"]["W"],
                  _b2(pee["l2"]["b"]), _b2(pee["ln"]["g"]), _b2(pee["ln"]["b"]))

    convs = params["convs"]
    pos_w = []
    for g in params["pos_upd"][:2]:
        pos_w += _gvp5(g)
    pos_w += [params["pos_upd"][2]["Wh"], params["pos_upd"][2]["Wu"]]
    nh = params["node_head"]
    head_w = [nh["l1"]["W"], _b2(nh["l1"]["b"]), nh["l2"]["W"],
              _b2(nh["l2"]["b"])]
    eh = params["edge_head"]
    ehead_w = [eh["l1"]["W"], _b2(eh["l1"]["b"]), eh["l2"]["W"],
               _b2(eh["l2"]["b"])]
    eupd_w = _eupd_weights(params["edge_upd"])

    us, ud = _gather(src1, dst1, u)
    m = _edge_msg(us, ud, e, *_msg_weights(convs[0]))
    pp = _scatter(dst3, m, zp)[0]
    u = _node_upd(u, pp, *_upd_weights(convs[0]))[0]

    us, ud = _gather(src1, dst1, u)
    m = _edge_msg(us, ud, e, *_msg_weights(convs[1]))
    pp = _scatter(dst3, m, zp)[0]
    u = _node_upd_pos(u, pp, *_upd_weights(convs[1]), *pos_w)[0]

    us, ud = _gather(src1, dst1, u)
    e = _edge_upd(us, ud, e, *eupd_w)[0]

    m = _edge_msg(us, ud, e, *_msg_weights(convs[2]))
    pp = _scatter(dst3, m, zp)[0]
    u = _node_upd(u, pp, *_upd_weights(convs[2]))[0]

    us, ud = _gather(src1, dst1, u)
    m = _edge_msg(us, ud, e, *_msg_weights(convs[3]))
    pp = _scatter(dst3, m, zp)[0]
    u, node_logits = _node_upd_pos_head(u, pp, *_upd_weights(convs[3]),
                                        *pos_w, *head_w)

    us, ud = _gather(src1, dst1, u)
    e, edge_logits = _edge_upd_head(us, ud, e, *eupd_w, *ehead_w)

    return node_logits, edge_logits, u[:, 112:115]


# speedup vs baseline: 11.0567x; 1.0460x over previous
import functools

import jax
import jax.numpy as jnp
from jax import lax
from jax.experimental import pallas as pl
from jax.experimental.pallas import tpu as pltpu
from jax.experimental.pallas import tpu_sc as plsc

NN = 10000
NNP = 10240
NE = 320000
NG = 64
NH = 64
NV = 16
NU = 128

NW = 32
PW = NE // NW
CH = 80
NCH = PW // CH
GB = 320
NBK = (PW - CH) // GB
GBS = 160
NBS = (PW - CH) // GBS
NPS = NNP // 16

BE = 2560
GE = NE // BE
BN = 2000
GN = NN // BN

F32 = jnp.float32


def _sig(x):
    return 1.0 / (1.0 + jnp.exp(-x))


def _silu(x):
    return x * _sig(x)


def _ln(x, g, b):
    m = jnp.mean(x, axis=-1, keepdims=True)
    v = jnp.mean((x - m) ** 2, axis=-1, keepdims=True)
    return (x - m) * lax.rsqrt(v + 1e-5) * g + b


def _dot(a, b):
    return jnp.dot(a, b, preferred_element_type=F32)


def _rbf_parts(xs, xd):
    diff = xd - xs
    d2 = jnp.sum(diff * diff, axis=-1, keepdims=True)
    dist = jnp.sqrt(d2 + 1e-8)
    inv = 1.0 / dist
    mu = (lax.broadcasted_iota(jnp.int32, (1, 16), 1).astype(F32)
          * (20.0 / 15.0))
    d = jnp.exp(-((dist - mu) ** 2) * (1.0 / (2.0 * 1.25 * 1.25)))
    xdp = [diff[:, k:k + 1] * inv for k in range(3)]
    return xdp, d


def _gvp_tail(z, Vh, wu, gate):
    s = _silu(z)
    mus = [_dot(h, wu) for h in Vh]
    if gate:
        gn = jnp.sqrt(mus[0] ** 2 + mus[1] ** 2 + mus[2] ** 2 + 1e-8)
        gt = _sig(gn)
        mus = [m * gt for m in mus]
    return s, mus


def _vn(Vh):
    return jnp.sqrt(Vh[0] ** 2 + Vh[1] ** 2 + Vh[2] ** 2 + 1e-8)


def _upd_gvp(s, V, wh, wu, ws, wv, b, gate=True):
    Vh = [_dot(v, wh) for v in V]
    z = _dot(s, ws) + _dot(_vn(Vh), wv) + b
    return _gvp_tail(z, Vh, wu, gate)


def _full_spec(shape):
    nd = len(shape)
    return pl.BlockSpec(shape, lambda i, _n=nd: (0,) * _n)


def _row_spec(block, width):
    return pl.BlockSpec((block, width), lambda i: (i, 0))


def _u_parts(u):
    s = u[:, 0:NH]
    V = [u[:, NH + 16 * p:NH + 16 * p + 16] for p in range(3)]
    x = u[:, 112:115]
    return s, V, x


def _pack_u(s, V, x):
    z = jnp.zeros((s.shape[0], 13), F32)
    return jnp.concatenate([s] + V + [x, z], axis=1)


def _make_node_emb():
    def body(a, c, nbi, t, x, w1a, w1t, w1c, b1, w2, b2, g, bl, out):
        oh = (nbi[...] == lax.broadcasted_iota(jnp.int32, (BN, NG), 1))
        tn = _dot(oh.astype(F32), t[...])
        h = _silu(_dot(a[...], w1a[...]) + tn * w1t[...] +
                  _dot(c[...], w1c[...]) + b1[...])
        h = _silu(_dot(h, w2[...]) + b2[...])
        s = _ln(h, g[...], bl[...])
        zv = jnp.zeros((BN, 16), F32)
        out[...] = _pack_u(s, [zv, zv, zv], x[...])

    return pl.pallas_call(
        body,
        grid=(GN,),
        in_specs=[
            _row_spec(BN, 16), _row_spec(BN, 6), _row_spec(BN, 1),
            _full_spec((NG, 1)), _row_spec(BN, 3),
            _full_spec((16, NH)), _full_spec((1, NH)), _full_spec((6, NH)),
            _full_spec((1, NH)), _full_spec((NH, NH)), _full_spec((1, NH)),
            _full_spec((1, NH)), _full_spec((1, NH)),
        ],
        out_specs=_row_spec(BN, NU),
        out_shape=jax.ShapeDtypeStruct((NN, NU), F32),
    )


def _make_edge_emb():
    def body(et, w1, b1, w2, b2, g, bl, out):
        h = _silu(_dot(et[...], w1[...]) + b1[...])
        h = _silu(_dot(h, w2[...]) + b2[...])
        out[...] = _ln(h, g[...], bl[...])

    return pl.pallas_call(
        body,
        grid=(GE,),
        in_specs=[
            _row_spec(BE, 5),
            _full_spec((5, NH)), _full_spec((1, NH)),
            _full_spec((NH, NH)), _full_spec((1, NH)),
            _full_spec((1, NH)), _full_spec((1, NH)),
        ],
        out_specs=_row_spec(BE, NH),
        out_shape=jax.ShapeDtypeStruct((NE, NH), F32),
    )


def _make_edge_msg():
    def body(*refs):
        i = iter(refs)
        us, ud, e = next(i), next(i), next(i)
        wh1, wu1 = next(i), next(i)
        w1a, w1b, w1e, w1d, w1v, b1 = (next(i) for _ in range(6))
        wh2, wu2, w2s, w2v, b2 = (next(i) for _ in range(5))
        wh3, wu3, w3s, w3v, b3 = (next(i) for _ in range(5))
        om = next(i)

        usr, udr = us[...], ud[...]
        ss, vsr, xs = _u_parts(usr)
        sd, _, xd = _u_parts(udr)
        xdp, d = _rbf_parts(xs, xd)
        wh1m = wh1[...]
        Vh = [_dot(vsr[p], wh1m[0:16, :]) + xdp[p] * wh1m[16:17, :]
              for p in range(3)]
        z = (_dot(ss, w1a[...]) + _dot(sd, w1b[...]) +
             _dot(e[...], w1e[...]) + _dot(d, w1d[...]) +
             _dot(_vn(Vh), w1v[...]) + b1[...])
        s, V = _gvp_tail(z, Vh, wu1[...], True)

        Vh = [_dot(v, wh2[...]) for v in V]
        z = _dot(s, w2s[...]) + _dot(_vn(Vh), w2v[...]) + b2[...]
        s, V = _gvp_tail(z, Vh, wu2[...], True)

        Vh = [_dot(v, wh3[...]) for v in V]
        z = _dot(s, w3s[...]) + _dot(_vn(Vh), w3v[...]) + b3[...]
        s, V = _gvp_tail(z, Vh, wu3[...], True)

        om[...] = jnp.concatenate([s] + V + [jnp.zeros((BE, 16), F32)],
                                  axis=1)

    wspecs = [
        _full_spec((17, 17)), _full_spec((17, NV)),
        _full_spec((NH, NH)), _full_spec((NH, NH)), _full_spec((NH, NH)),
        _full_spec((16, NH)), _full_spec((17, NH)), _full_spec((1, NH)),
        _full_spec((NV, NV)), _full_spec((NV, NV)),
        _full_spec((NH, NH)), _full_spec((NV, NH)), _full_spec((1, NH)),
        _full_spec((NV, NV)), _full_spec((NV, NV)),
        _full_spec((NH, NH)), _full_spec((NV, NH)), _full_spec((1, NH)),
    ]
    return pl.pallas_call(
        body,
        grid=(GE,),
        in_specs=[_row_spec(BE, NU), _row_spec(BE, NU),
                  _row_spec(BE, NH)] + wspecs,
        out_specs=_row_spec(BE, NU),
        out_shape=jax.ShapeDtypeStruct((NE, NU), F32),
    )


def _make_node_upd(pos, head):
    def body(*refs):
        i = iter(refs)
        u0, pp = next(i), next(i)
        g1, bl1, g2, bl2 = (next(i) for _ in range(4))
        uw = [[next(i) for _ in range(5)] for _ in range(3)]
        if pos:
            pw = [[next(i) for _ in range(5)] for _ in range(2)]
            wh3, wu3 = next(i), next(i)
        if head:
            hw1, hb1, hw2, hb2 = (next(i) for _ in range(4))
        ou = next(i)
        ol = next(i) if head else None

        u0r = u0[...]
        s0, V0, x = _u_parts(u0r)
        ppr = pp[...]
        agg = (ppr[0] + ppr[1]) * (1.0 / 100.0)
        s = _ln(s0 + agg[:, 0:NH], g1[...], bl1[...])
        V = [V0[p] + agg[:, NH + 16 * p:NH + 16 * p + 16] for p in range(3)]
        us, uv = s, V
        for w in uw:
            us, uv = _upd_gvp(us, uv, w[0][...], w[1][...], w[2][...],
                              w[3][...], w[4][...])
        s2 = _ln(s + us, g2[...], bl2[...])
        V2 = [V[p] + uv[p] for p in range(3)]

        if pos:
            qs, qv = s2, V2
            for w in pw:
                qs, qv = _upd_gvp(qs, qv, w[0][...], w[1][...], w[2][...],
                                  w[3][...], w[4][...])
            Vh = [_dot(v, wh3[...]) for v in qv]
            mus = [_dot(h, wu3[...]) for h in Vh]
            x = jnp.concatenate([x[:, p:p + 1] + mus[p] for p in range(3)],
                                axis=1)
        ou[...] = _pack_u(s2, V2, x)
        if head:
            hh = _silu(_dot(s2, hw1[...]) + hb1[...])
            ol[...] = _dot(hh, hw2[...]) + hb2[...]

    specs = [_row_spec(BN, NU),
             pl.BlockSpec((2, BN, NU), lambda i: (0, i, 0)),
             _full_spec((1, NH)), _full_spec((1, NH)),
             _full_spec((1, NH)), _full_spec((1, NH))]
    gvp_w = [_full_spec((NV, NV)), _full_spec((NV, NV)),
             _full_spec((NH, NH)), _full_spec((NV, NH)), _full_spec((1, NH))]
    specs += gvp_w * 3
    if pos:
        specs += gvp_w * 2
        specs += [_full_spec((NV, NV)), _full_spec((NV, 1))]
    if head:
        specs += [_full_spec((NH, NH)), _full_spec((1, NH)),
                  _full_spec((NH, 22)), _full_spec((1, 22))]
    outs = [_row_spec(BN, NU)]
    oshapes = [jax.ShapeDtypeStruct((NN, NU), F32)]
    if head:
        outs.append(_row_spec(BN, 22))
        oshapes.append(jax.ShapeDtypeStruct((NN, 22), F32))
    return pl.pallas_call(body, grid=(GN,), in_specs=specs,
                          out_specs=outs, out_shape=oshapes)


def _make_edge_upd(head):
    def body(*refs):
        i = iter(refs)
        us, ud, e = (next(i) for _ in range(3))
        wa, wb, we, wd, b1, w2, b2, g, bl = (next(i) for _ in range(9))
        if head:
            hw1, hb1, hw2, hb2 = (next(i) for _ in range(4))
        oe = next(i)
        ol = next(i) if head else None

        ss, _, xs = _u_parts(us[...])
        sd, _, xd = _u_parts(ud[...])
        _, d = _rbf_parts(xs, xd)
        er = e[...]
        h = _silu(_dot(ss, wa[...]) + _dot(sd, wb[...]) +
                  _dot(er, we[...]) + _dot(d, wd[...]) + b1[...])
        h = _silu(_dot(h, w2[...]) + b2[...])
        en = _ln(er + h, g[...], bl[...])
        oe[...] = en
        if head:
            hh = _silu(_dot(en, hw1[...]) + hb1[...])
            ol[...] = _dot(hh, hw2[...]) + hb2[...]

    specs = [_row_spec(BE, NU), _row_spec(BE, NU), _row_spec(BE, NH),
             _full_spec((NH, NH)), _full_spec((NH, NH)), _full_spec((NH, NH)),
             _full_spec((16, NH)), _full_spec((1, NH)),
             _full_spec((NH, NH)), _full_spec((1, NH)),
             _full_spec((1, NH)), _full_spec((1, NH))]
    outs = [_row_spec(BE, NH)]
    oshapes = [jax.ShapeDtypeStruct((NE, NH), F32)]
    if head:
        specs += [_full_spec((NH, NH)), _full_spec((1, NH)),
                  _full_spec((NH, 5)), _full_spec((1, 5))]
        outs.append(_row_spec(BE, 5))
        oshapes.append(jax.ShapeDtypeStruct((NE, 5), F32))
    return pl.pallas_call(body, grid=(GE,), in_specs=specs,
                          out_specs=outs, out_shape=oshapes)


def _sc_mesh():
    return plsc.VectorSubcoreMesh(core_axis_name="c", subcore_axis_name="s",
                                  num_cores=2)


def _make_gather():
    out_type = [jax.ShapeDtypeStruct((NE, NU), F32),
                jax.ShapeDtypeStruct((NE, NU), F32)]
    scratch = [pltpu.VMEM((PW,), jnp.int32), pltpu.VMEM((PW,), jnp.int32),
               pltpu.VMEM((GB, NU), F32), pltpu.VMEM((GB, NU), F32),
               pltpu.SemaphoreType.DMA]

    @functools.partial(pl.kernel, mesh=_sc_mesh(), out_type=out_type,
                       scratch_types=scratch)
    def k(src_h, dst_h, u_h, o_us, o_ud, isrc, idst, b1, b2, sem):
        wid = lax.axis_index("s") * 2 + lax.axis_index("c")
        pltpu.sync_copy(src_h.at[pl.ds(wid * PW, PW)], isrc)
        pltpu.sync_copy(dst_h.at[pl.ds(wid * PW, PW)], idst)

        def it(j, carry):
            off = wid * PW + j * GB
            cps = []
            for q in range(GB // CH):
                ia = isrc.at[pl.ds(j * GB + q * CH, CH)]
                ib = idst.at[pl.ds(j * GB + q * CH, CH)]
                cps.append(pltpu.async_copy(
                    u_h.at[ia], b1.at[pl.ds(q * CH, CH)], sem))
                cps.append(pltpu.async_copy(
                    u_h.at[ib], b2.at[pl.ds(q * CH, CH)], sem))
            for cp in cps:
                cp.wait()
            pltpu.sync_copy(b1, o_us.at[pl.ds(off, GB)])
            pltpu.sync_copy(b2, o_ud.at[pl.ds(off, GB)])
            return carry

        lax.fori_loop(0, NBK, it, 0)
        toff = wid * PW + NBK * GB
        ia = isrc.at[pl.ds(NBK * GB, CH)]
        ib = idst.at[pl.ds(NBK * GB, CH)]
        c1 = pltpu.async_copy(u_h.at[ia], b1.at[pl.ds(0, CH)], sem)
        c2 = pltpu.async_copy(u_h.at[ib], b2.at[pl.ds(0, CH)], sem)
        c1.wait()
        c2.wait()
        pltpu.sync_copy(b1.at[pl.ds(0, CH)], o_us.at[pl.ds(toff, CH)])
        pltpu.sync_copy(b2.at[pl.ds(0, CH)], o_ud.at[pl.ds(toff, CH)])

    return k


def _make_scatter():
    out_type = [jax.ShapeDtypeStruct((2, NNP, NU), F32)]
    scratch = [pltpu.VMEM((NCH, CH), jnp.int32),
               pltpu.VMEM((GBS, NU), F32),
               pltpu.VMEM_SHARED((NNP, NU), F32)]

    @functools.partial(pl.kernel, mesh=_sc_mesh(), out_type=out_type,
                       scratch_types=scratch)
    def k(dst_h, m_h, z_h, o_p, idxb, bm, acc):
        cid = lax.axis_index("c")
        sid = lax.axis_index("s")
        wid = sid * 2 + cid
        pltpu.sync_copy(z_h.at[pl.ds(sid * NPS, NPS)],
                        acc.at[pl.ds(sid * NPS, NPS)])
        pltpu.sync_copy(dst_h.at[wid], idxb)
        plsc.subcore_barrier()

        def it(j, carry):
            off = wid * PW + j * GBS
            pltpu.sync_copy(m_h.at[pl.ds(off, GBS)], bm)
            for q in range(GBS // CH):
                pltpu.sync_copy(bm.at[pl.ds(q * CH, CH)],
                                acc.at[idxb.at[j * (GBS // CH) + q]],
                                add=True)
            return carry

        lax.fori_loop(0, NBS, it, 0)
        toff = wid * PW + NBS * GBS
        pltpu.sync_copy(m_h.at[pl.ds(toff, CH)], bm.at[pl.ds(0, CH)])
        pltpu.sync_copy(bm.at[pl.ds(0, CH)], acc.at[idxb.at[NCH - 1]],
                        add=True)
        plsc.subcore_barrier()
        pltpu.sync_copy(acc.at[pl.ds(sid * NPS, NPS)],
                        o_p.at[cid, pl.ds(sid * NPS, NPS)])

    return k


_node_emb = _make_node_emb()
_edge_emb = _make_edge_emb()
_edge_msg = _make_edge_msg()
_node_upd = _make_node_upd(False, False)
_node_upd_pos = _make_node_upd(True, False)
_node_upd_pos_head = _make_node_upd(True, True)
_edge_upd = _make_edge_upd(False)
_edge_upd_head = _make_edge_upd(True)

_sc_kernels = {}


def _sc_get(name, factory):
    if name not in _sc_kernels:
        _sc_kernels[name] = factory()
    return _sc_kernels[name]


def _gather(*args):
    return _sc_get("gather", _make_gather)(*args)


def _scatter(*args):
    return _sc_get("scatter", _make_scatter)(*args)


def _b2(b):
    return b.reshape(1, -1)


def _msg_weights(conv):
    g1, g2, g3 = conv["msg"]
    w1 = g1["Ws"]["W"]
    out = [g1["Wh"], g1["Wu"],
           w1[0:64], w1[64:128], w1[128:192], w1[192:208], w1[208:225],
           _b2(g1["Ws"]["b"])]
    for g in (g2, g3):
        w = g["Ws"]["W"]
        out += [g["Wh"], g["Wu"], w[0:64], w[64:80], _b2(g["Ws"]["b"])]
    return out


def _gvp5(g):
    w = g["Ws"]["W"]
    return [g["Wh"], g["Wu"], w[0:64], w[64:80], _b2(g["Ws"]["b"])]


def _upd_weights(conv):
    out = [_b2(conv["ln1"]["g"]), _b2(conv["ln1"]["b"]),
           _b2(conv["ln2"]["g"]), _b2(conv["ln2"]["b"])]
    for g in conv["upd"]:
        out += _gvp5(g)
    return out


def _eupd_weights(p):
    w = p["l1"]["W"]
    return [w[0:64], w[64:128], w[128:192], w[192:208], _b2(p["l1"]["b"]),
            p["l2"]["W"], _b2(p["l2"]["b"]),
            _b2(p["ln"]["g"]), _b2(p["ln"]["b"])]


def kernel(a_t, c_t, x_t, e_t, t, edge_index, node_batch_idx,
           upper_edge_mask, params):
    src1 = edge_index[0]
    dst1 = edge_index[1]
    dst3 = dst1.reshape(NW, NCH, CH)
    nbi = node_batch_idx.reshape(NN, 1)
    t2 = t.reshape(NG, 1)
    zp = jnp.zeros((NNP, NU), F32)

    pe = params["scalar_emb"]
    w1 = pe["l1"]["W"]
    u = _node_emb(a_t, c_t, nbi, t2, x_t, w1[0:16], w1[16:17], w1[17:23],
                  _b2(pe["l1"]["b"]), pe["l2"]["W"], _b2(pe["l2"]["b"]),
                  _b2(pe["ln"]["g"]), _b2(pe["ln"]["b"]))
    pee = params["edge_emb"]
    e = _edge_emb(e_t, pee["l1"]["W"], _b2(pee["l1"]["b"]), pee["l2"]["W"],
                  _b2(pee["l2"]["b"]), _b2(pee["ln"]["g"]), _b2(pee["ln"]["b"]))

    convs = params["convs"]
    pos_w = []
    for g in params["pos_upd"][:2]:
        pos_w += _gvp5(g)
    pos_w += [params["pos_upd"][2]["Wh"], params["pos_upd"][2]["Wu"]]
    nh = params["node_head"]
    head_w = [nh["l1"]["W"], _b2(nh["l1"]["b"]), nh["l2"]["W"],
              _b2(nh["l2"]["b"])]
    eh = params["edge_head"]
    ehead_w = [eh["l1"]["W"], _b2(eh["l1"]["b"]), eh["l2"]["W"],
               _b2(eh["l2"]["b"])]
    eupd_w = _eupd_weights(params["edge_upd"])

    us, ud = _gather(src1, dst1, u)
    m = _edge_msg(us, ud, e, *_msg_weights(convs[0]))
    pp = _scatter(dst3, m, zp)[0]
    u = _node_upd(u, pp, *_upd_weights(convs[0]))[0]

    us, ud = _gather(src1, dst1, u)
    m = _edge_msg(us, ud, e, *_msg_weights(convs[1]))
    pp = _scatter(dst3, m, zp)[0]
    u = _node_upd_pos(u, pp, *_upd_weights(convs[1]), *pos_w)[0]

    us, ud = _gather(src1, dst1, u)
    e = _edge_upd(us, ud, e, *eupd_w)[0]

    m = _edge_msg(us, ud, e, *_msg_weights(convs[2]))
    pp = _scatter(dst3, m, zp)[0]
    u = _node_upd(u, pp, *_upd_weights(convs[2]))[0]

    us, ud = _gather(src1, dst1, u)
    m = _edge_msg(us, ud, e, *_msg_weights(convs[3]))
    pp = _scatter(dst3, m, zp)[0]
    u, node_logits = _node_upd_pos_head(u, pp, *_upd_weights(convs[3]),
                                        *pos_w, *head_w)

    us, ud = _gather(src1, dst1, u)
    e, edge_logits = _edge_upd_head(us, ud, e, *eupd_w, *ehead_w)

    return node_logits, edge_logits, u[:, 112:115]
